# Initial kernel scaffold; baseline (speedup 1.0000x reference)
import jax
import jax.numpy as jnp
from jax.experimental import pallas as pl


def kernel(x, neigh_sorted_orders, W, b):
    raise NotImplementedError("write your pallas kernel here")


# speedup vs baseline: 1.0177x; 1.0177x over previous
import functools

import jax
import jax.numpy as jnp
from jax import lax
from jax.experimental import pallas as pl
from jax.experimental.pallas import tpu as pltpu
from jax.experimental.pallas import tpu_sc as plsc

N = 10000
D = 128
K = 32
F = K * D
NW = 32
NPAD = 10240
BPW = NPAD // NW
CHUNK = 128
NODES_PER_CHUNK = CHUNK // K
NCHUNK = BPW * K // CHUNK


def _matmul_body(x_ref, w_ref, b_ref, y_ref):
    y_ref[...] = (
        jnp.dot(x_ref[...], w_ref[...], preferred_element_type=jnp.float32)
        + b_ref[...]
    )


def _tc_matmul(x, wr, bpad):
    MB, NB = 1000, 1024
    return pl.pallas_call(
        _matmul_body,
        grid=(N // MB, F // NB),
        in_specs=[
            pl.BlockSpec((MB, D), lambda i, j: (i, 0)),
            pl.BlockSpec((D, NB), lambda i, j: (0, j)),
            pl.BlockSpec((1, NB), lambda i, j: (0, j)),
        ],
        out_specs=pl.BlockSpec((MB, NB), lambda i, j: (i, j)),
        out_shape=jax.ShapeDtypeStruct((N, F), jnp.float32),
        compiler_params=pltpu.CompilerParams(
            dimension_semantics=("parallel", "parallel")
        ),
    )(x, wr, bpad)


def _sc_bag(y4_hbm, fidx_hbm, out_hbm, idx_v, stag_v, out_v, sem):
    wid = lax.axis_index("s") * 2 + lax.axis_index("c")
    pltpu.sync_copy(fidx_hbm.at[wid], idx_v)

    def chunk_body(c, carry):
        pltpu.async_copy(y4_hbm.at[idx_v.at[c]], stag_v, sem).wait()
        for node in range(NODES_PER_CHUNK):
            base = node * K
            for l in range(D // 16):
                sl = pl.ds(l * 16, 16)
                acc = stag_v[base, sl]
                for k in range(1, K):
                    acc = acc + stag_v[base + k, sl]
                out_v[c * NODES_PER_CHUNK + node, sl] = acc
        return carry

    lax.fori_loop(0, NCHUNK, chunk_body, 0)
    pltpu.sync_copy(out_v, out_hbm.at[pl.ds(wid * BPW, BPW)])


@functools.cache
def _sc_bag_call():
    return pl.kernel(
        _sc_bag,
        out_type=jax.ShapeDtypeStruct((NPAD, D), jnp.float32),
        mesh=plsc.VectorSubcoreMesh(core_axis_name="c", subcore_axis_name="s"),
        scratch_types=[
            pltpu.VMEM((NCHUNK, CHUNK), jnp.int32),
            pltpu.VMEM((CHUNK, D), jnp.float32),
            pltpu.VMEM((BPW, D), jnp.float32),
            pltpu.SemaphoreType.DMA,
        ],
    )


def kernel(x, neigh_sorted_orders, W, b):
    idx = neigh_sorted_orders.astype(jnp.int32)
    wr = W.reshape(D, K, D).transpose(2, 1, 0).reshape(D, F)
    bpad = jnp.zeros((1, F), jnp.float32).at[0, :D].set(b)
    y = _tc_matmul(x, wr, bpad)
    y4 = y.reshape(N * K, D)
    fidx = idx * K + jnp.arange(K, dtype=jnp.int32)[None, :]
    fidx = jnp.pad(fidx, ((0, NPAD - N), (0, 0)))
    fidx = fidx.reshape(NW, NCHUNK, CHUNK)
    out = _sc_bag_call()(y4, fidx)
    return out[:N]


# speedup vs baseline: 1.2060x; 1.1850x over previous
import functools

import jax
import jax.numpy as jnp
from jax import lax
from jax.experimental import pallas as pl
from jax.experimental.pallas import tpu as pltpu
from jax.experimental.pallas import tpu_sc as plsc

N = 10000
D = 128
K = 32
F = K * D
NW = 32
NPAD = 10240
BPW = NPAD // NW
CHUNK = 128
NODES_PER_CHUNK = CHUNK // K
NCHUNK = BPW * K // CHUNK


def _matmul_body(x_ref, w_ref, b_ref, y_ref):
    y_ref[...] = (
        jnp.dot(x_ref[...], w_ref[...], preferred_element_type=jnp.float32)
        + b_ref[...]
    )


def _tc_matmul(x, wr, bpad):
    MB, NB = 1000, 1024
    return pl.pallas_call(
        _matmul_body,
        grid=(N // MB, F // NB),
        in_specs=[
            pl.BlockSpec((MB, D), lambda i, j: (i, 0)),
            pl.BlockSpec((D, NB), lambda i, j: (0, j)),
            pl.BlockSpec((1, NB), lambda i, j: (0, j)),
        ],
        out_specs=pl.BlockSpec((MB, NB), lambda i, j: (i, j)),
        out_shape=jax.ShapeDtypeStruct((N, F), jnp.float32),
        compiler_params=pltpu.CompilerParams(
            dimension_semantics=("parallel", "parallel")
        ),
    )(x, wr, bpad)


GCH = 64
NGC = BPW // GCH


def _sc_bag(y4_hbm, fidx_hbm, out_hbm, idx_v, out_v, sem):
    wid = lax.axis_index("s") * 2 + lax.axis_index("c")
    pltpu.sync_copy(fidx_hbm.at[wid], idx_v)

    for c in range(NGC):
        sl = pl.ds(c * GCH, GCH)
        pltpu.async_copy(y4_hbm.at[idx_v.at[0, sl]], out_v.at[sl], sem).wait()

    def k_body(k, carry):
        for c in range(NGC):
            sl = pl.ds(c * GCH, GCH)
            pltpu.async_copy(
                y4_hbm.at[idx_v.at[k, sl]], out_v.at[sl], sem, add=True
            ).wait()
        return carry

    lax.fori_loop(1, K, k_body, 0)
    pltpu.sync_copy(out_v, out_hbm.at[pl.ds(wid * BPW, BPW)])


@functools.cache
def _sc_bag_call():
    return pl.kernel(
        _sc_bag,
        out_type=jax.ShapeDtypeStruct((NPAD, D), jnp.float32),
        mesh=plsc.VectorSubcoreMesh(core_axis_name="c", subcore_axis_name="s"),
        scratch_types=[
            pltpu.VMEM((K, BPW), jnp.int32),
            pltpu.VMEM((BPW, D), jnp.float32),
            pltpu.SemaphoreType.DMA,
        ],
    )


def kernel(x, neigh_sorted_orders, W, b):
    idx = neigh_sorted_orders.astype(jnp.int32)
    wr = W.reshape(D, K, D).transpose(2, 1, 0).reshape(D, F)
    bpad = jnp.zeros((1, F), jnp.float32).at[0, :D].set(b)
    y = _tc_matmul(x, wr, bpad)
    y4 = y.reshape(N * K, D)
    fidx = idx * K + jnp.arange(K, dtype=jnp.int32)[None, :]
    fidx = jnp.pad(fidx, ((0, NPAD - N), (0, 0)))
    fidx = fidx.reshape(NW, BPW, K).transpose(0, 2, 1)
    out = _sc_bag_call()(y4, fidx)
    return out[:N]


# speedup vs baseline: 1.3595x; 1.1273x over previous
import functools

import jax
import jax.numpy as jnp
from jax import lax
from jax.experimental import pallas as pl
from jax.experimental.pallas import tpu as pltpu
from jax.experimental.pallas import tpu_sc as plsc

N = 10000
D = 128
K = 32
F = K * D
NW = 32
NPAD = 10240
BPW = NPAD // NW
CHUNK = 128
NODES_PER_CHUNK = CHUNK // K
NCHUNK = BPW * K // CHUNK


def _matmul_body(x_ref, w_ref, b_ref, y_ref):
    y_ref[...] = (
        jnp.dot(x_ref[...], w_ref[...], preferred_element_type=jnp.float32)
        + b_ref[...]
    )


def _tc_matmul(x, wr, bpad):
    MB, NB = 1000, 1024
    return pl.pallas_call(
        _matmul_body,
        grid=(N // MB, F // NB),
        in_specs=[
            pl.BlockSpec((MB, D), lambda i, j: (i, 0)),
            pl.BlockSpec((D, NB), lambda i, j: (0, j)),
            pl.BlockSpec((1, NB), lambda i, j: (0, j)),
        ],
        out_specs=pl.BlockSpec((MB, NB), lambda i, j: (i, j)),
        out_shape=jax.ShapeDtypeStruct((N, F), jnp.float32),
        compiler_params=pltpu.CompilerParams(
            dimension_semantics=("parallel", "parallel")
        ),
    )(x, wr, bpad)


GCH = 64
NGC = BPW // GCH


def _sc_bag(y4_hbm, fidx_hbm, out_hbm, idx_v, out_v, sem):
    wid = lax.axis_index("s") * 2 + lax.axis_index("c")
    pltpu.sync_copy(fidx_hbm.at[wid], idx_v)

    def fire(k, add):
        for c in range(NGC):
            sl = pl.ds(c * GCH, GCH)
            pltpu.async_copy(
                y4_hbm.at[idx_v.at[k, sl]], out_v.at[sl], sem, add=add
            )

    def drain():
        for c in range(NGC):
            sl = pl.ds(0, GCH)
            pltpu.make_async_copy(y4_hbm.at[sl], out_v.at[sl], sem).wait()

    fire(0, False)
    drain()
    fire(1, True)

    def k_body(k, carry):
        fire(k, True)
        drain()
        return carry

    lax.fori_loop(2, K, k_body, 0)
    drain()
    pltpu.sync_copy(out_v, out_hbm.at[pl.ds(wid * BPW, BPW)])


@functools.cache
def _sc_bag_call():
    return pl.kernel(
        _sc_bag,
        out_type=jax.ShapeDtypeStruct((NPAD, D), jnp.float32),
        mesh=plsc.VectorSubcoreMesh(core_axis_name="c", subcore_axis_name="s"),
        scratch_types=[
            pltpu.VMEM((K, BPW), jnp.int32),
            pltpu.VMEM((BPW, D), jnp.float32),
            pltpu.SemaphoreType.DMA,
        ],
    )


def kernel(x, neigh_sorted_orders, W, b):
    idx = neigh_sorted_orders.astype(jnp.int32)
    wr = W.reshape(D, K, D).transpose(2, 1, 0).reshape(D, F)
    bpad = jnp.zeros((1, F), jnp.float32).at[0, :D].set(b)
    y = _tc_matmul(x, wr, bpad)
    y4 = y.reshape(N * K, D)
    fidx = idx * K + jnp.arange(K, dtype=jnp.int32)[None, :]
    fidx = jnp.pad(fidx, ((0, NPAD - N), (0, 0)))
    fidx = fidx.reshape(NW, BPW, K).transpose(0, 2, 1)
    out = _sc_bag_call()(y4, fidx)
    return out[:N]


# speedup vs baseline: 1.3623x; 1.0020x over previous
import functools

import jax
import jax.numpy as jnp
from jax import lax
from jax.experimental import pallas as pl
from jax.experimental.pallas import tpu as pltpu
from jax.experimental.pallas import tpu_sc as plsc

N = 10000
D = 128
K = 32
F = K * D
NW = 32
NPAD = 10240
BPW = NPAD // NW
CHUNK = 128
NODES_PER_CHUNK = CHUNK // K
NCHUNK = BPW * K // CHUNK


def _matmul_body(x_ref, w_ref, b_ref, y_ref):
    y_ref[...] = (
        jnp.dot(x_ref[...], w_ref[...], preferred_element_type=jnp.float32)
        + b_ref[...]
    )


def _tc_matmul(x, wr, bpad):
    MB, NB = 1000, 1024
    return pl.pallas_call(
        _matmul_body,
        grid=(N // MB, F // NB),
        in_specs=[
            pl.BlockSpec((MB, D), lambda i, j: (i, 0)),
            pl.BlockSpec((D, NB), lambda i, j: (0, j)),
            pl.BlockSpec((1, NB), lambda i, j: (0, j)),
        ],
        out_specs=pl.BlockSpec((MB, NB), lambda i, j: (i, j)),
        out_shape=jax.ShapeDtypeStruct((N, F), jnp.float32),
        compiler_params=pltpu.CompilerParams(
            dimension_semantics=("parallel", "parallel")
        ),
    )(x, wr, bpad)


GCHUNKS = ((0, 128), (128, 128), (256, 64))


def _sc_bag(y4_hbm, fidx_hbm, out_hbm, idx_v, out_v, sem):
    wid = lax.axis_index("s") * 2 + lax.axis_index("c")
    pltpu.sync_copy(fidx_hbm.at[wid], idx_v)

    def fire(k, add):
        for off, sz in GCHUNKS:
            sl = pl.ds(off, sz)
            pltpu.async_copy(
                y4_hbm.at[idx_v.at[k, sl]], out_v.at[sl], sem, add=add
            )

    def drain():
        for off, sz in GCHUNKS:
            sl = pl.ds(0, sz)
            pltpu.make_async_copy(y4_hbm.at[sl], out_v.at[sl], sem).wait()

    fire(0, False)
    drain()
    fire(1, True)
    fire(2, True)

    def k_body(k, carry):
        fire(k, True)
        drain()
        return carry

    lax.fori_loop(3, K, k_body, 0)
    drain()
    drain()
    pltpu.sync_copy(out_v, out_hbm.at[pl.ds(wid * BPW, BPW)])


@functools.cache
def _sc_bag_call():
    return pl.kernel(
        _sc_bag,
        out_type=jax.ShapeDtypeStruct((NPAD, D), jnp.float32),
        mesh=plsc.VectorSubcoreMesh(core_axis_name="c", subcore_axis_name="s"),
        scratch_types=[
            pltpu.VMEM((K, BPW), jnp.int32),
            pltpu.VMEM((BPW, D), jnp.float32),
            pltpu.SemaphoreType.DMA,
        ],
    )


def kernel(x, neigh_sorted_orders, W, b):
    idx = neigh_sorted_orders.astype(jnp.int32)
    wr = W.reshape(D, K, D).transpose(2, 1, 0).reshape(D, F)
    bpad = jnp.zeros((1, F), jnp.float32).at[0, :D].set(b)
    y = _tc_matmul(x, wr, bpad)
    y4 = y.reshape(N * K, D)
    fidx = idx * K + jnp.arange(K, dtype=jnp.int32)[None, :]
    fidx = jnp.pad(fidx, ((0, NPAD - N), (0, 0)))
    fidx = fidx.reshape(NW, BPW, K).transpose(0, 2, 1)
    out = _sc_bag_call()(y4, fidx)
    return out[:N]
